# baseline (speedup 1.0000x reference)
import functools

import jax
import jax.numpy as jnp
from jax import lax
from jax.experimental import pallas as pl
from jax.experimental.pallas import tpu as pltpu
from jax.experimental.pallas import tpu_sc as plsc

NC = 2
NS = 16
NW = NC * NS
BW = 128
L = 16


def _transpose_into(gbuf, tbuf, d):
    def body(f, carry):
        fv = jnp.zeros((L,), jnp.int32) + f
        for k in range(BW // L):
            rows = lax.iota(jnp.int32, L) + (k * L)
            v = plsc.load_gather(gbuf, [rows, fv])
            tbuf[f, pl.ds(k * L, L)] = v
        return carry

    lax.fori_loop(0, d, body, 0)


def _emb_kernel(b, h, d):
    mesh = plsc.VectorSubcoreMesh(core_axis_name="c", subcore_axis_name="s")

    @functools.partial(
        pl.kernel,
        mesh=mesh,
        compiler_params=pltpu.CompilerParams(
            use_tc_tiling_on_sc=False, needs_layout_passes=False
        ),
        out_type=jax.ShapeDtypeStruct((h, d, b), jnp.float32),
        scratch_types=[
            pltpu.VMEM((h, BW), jnp.int32),
            pltpu.VMEM((2, BW, d), jnp.float32),
            pltpu.VMEM((2, d, BW), jnp.float32),
            pltpu.SemaphoreType.DMA,
            pltpu.SemaphoreType.DMA,
        ],
    )
    def k(xt_hbm, table_hbm, out_hbm, idx_v, gbuf, tbuf, sem0, sem1):
        wid = lax.axis_index("s") * NC + lax.axis_index("c")
        b0 = wid * BW
        pltpu.sync_copy(xt_hbm.at[:, pl.ds(b0, BW)], idx_v)
        sems = (sem0, sem1)

        def fire(hh, buf):
            pltpu.async_copy(table_hbm.at[idx_v.at[hh]], gbuf.at[buf], sems[buf])

        def drain(hh, buf):
            pltpu.make_async_copy(
                table_hbm.at[idx_v.at[hh]], gbuf.at[buf], sems[buf]
            ).wait()

        def emit(hh, buf):
            drain(hh, buf)
            _transpose_into(gbuf.at[buf], tbuf.at[buf], d)
            pltpu.sync_copy(tbuf.at[buf], out_hbm.at[hh, :, pl.ds(b0, BW)])

        fire(0, 0)
        fire(1, 1)

        def body(i, carry):
            h2 = 2 * i
            emit(h2, 0)
            fire(h2 + 2, 0)
            emit(h2 + 1, 1)
            fire(h2 + 3, 1)
            return carry

        lax.fori_loop(0, h // 2 - 1, body, 0)
        emit(h - 2, 0)
        emit(h - 1, 1)

    return k


def kernel(x, table):
    b, h = x.shape
    v, d = table.shape
    assert b % NW == 0 and b // NW == BW and h % 2 == 0
    xt = jnp.transpose(jnp.asarray(x, jnp.int32))
    outt = _emb_kernel(b, h, d)(xt, table)
    return jnp.transpose(outt, (2, 0, 1))
